# Initial kernel scaffold; baseline (speedup 1.0000x reference)
import jax
import jax.numpy as jnp
from jax.experimental import pallas as pl


def kernel(h, edge_index):
    raise NotImplementedError("write your pallas kernel here")


# speedup vs baseline: 2.0935x; 2.0935x over previous
import functools

import jax
import jax.numpy as jnp
from jax import lax
from jax.experimental import pallas as pl
from jax.experimental.pallas import tpu as pltpu
from jax.experimental.pallas import tpu_sc as plsc

_GDN = lax.GatherDimensionNumbers(
    offset_dims=(), collapsed_slice_dims=(0,), start_index_map=(0,))


def _lane_shuffle(x, idx):
    return lax.gather(x, idx[:, None], dimension_numbers=_GDN,
                      slice_sizes=(1,),
                      mode=lax.GatherScatterMode.PROMISE_IN_BOUNDS)


N_NODES = 10000
N_EDGES = 320000
D = 128
L = 16
C = 80


def _edge_dot(h, src, dst):
    info = plsc.get_sparse_core_info()
    nc, ns = info.num_cores, info.num_subcores
    nw = nc * ns
    ew = N_EDGES // nw
    n_chunks = ew // C

    @functools.partial(
        pl.kernel,
        out_type=jax.ShapeDtypeStruct((N_EDGES,), jnp.float32),
        mesh=plsc.VectorSubcoreMesh(core_axis_name="c", subcore_axis_name="s"),
        scratch_types=[
            pltpu.VMEM((C,), jnp.int32),
            pltpu.VMEM((C,), jnp.int32),
            pltpu.VMEM((C, D), jnp.float32),
            pltpu.VMEM((C, D), jnp.float32),
            pltpu.VMEM((C,), jnp.float32),
            pltpu.SemaphoreType.DMA,
        ],
    )
    def k(h_ref, src_ref, dst_ref, out_ref,
          idx_s, idx_d, rows_s, rows_d, scores, sem):
        wid = lax.axis_index("s") * nc + lax.axis_index("c")
        lane = lax.iota(jnp.int32, L)
        rot = [(lane + k) & (L - 1) for k in (8, 4, 2, 1)]

        def body(g, carry):
            base = pl.multiple_of(wid * ew + g * C, 8)
            pltpu.sync_copy(src_ref.at[pl.ds(base, C)], idx_s)
            pltpu.sync_copy(dst_ref.at[pl.ds(base, C)], idx_d)
            cp_s = pltpu.async_copy(h_ref.at[idx_s], rows_s, sem)
            cp_d = pltpu.async_copy(h_ref.at[idx_d], rows_d, sem)
            cp_s.wait()
            cp_d.wait()
            for eg in range(C // L):
                acc = jnp.zeros((L,), jnp.float32)
                for e in range(L):
                    ei = eg * L + e
                    p = rows_s[ei, pl.ds(0, L)] * rows_d[ei, pl.ds(0, L)]
                    for j in range(1, D // L):
                        p = p + (rows_s[ei, pl.ds(j * L, L)]
                                 * rows_d[ei, pl.ds(j * L, L)])
                    for r in rot:
                        p = p + _lane_shuffle(p, r)
                    acc = jnp.where(lane == e, p, acc)
                scores[pl.ds(eg * L, L)] = acc
            pltpu.sync_copy(scores, out_ref.at[pl.ds(base, C)])
            return carry

        lax.fori_loop(0, n_chunks, body, 0)

    return k(h, src, dst)


def kernel(h, edge_index):
    ei = edge_index.astype(jnp.int32)
    scores = _edge_dot(h, ei[0], ei[1])
    return scores.reshape(N_EDGES, 1)


# speedup vs baseline: 3.6578x; 1.7472x over previous
import functools

import jax
import jax.numpy as jnp
from jax import lax
from jax.experimental import pallas as pl
from jax.experimental.pallas import tpu as pltpu
from jax.experimental.pallas import tpu_sc as plsc

_GDN = lax.GatherDimensionNumbers(
    offset_dims=(), collapsed_slice_dims=(0,), start_index_map=(0,))


def _lane_shuffle(x, idx):
    return lax.gather(x, idx[:, None], dimension_numbers=_GDN,
                      slice_sizes=(1,),
                      mode=lax.GatherScatterMode.PROMISE_IN_BOUNDS)


N_NODES = 10000
N_EDGES = 320000
D = 128
L = 16
C = 80


def _edge_dot(h, src, dst):
    info = plsc.get_sparse_core_info()
    nc, ns = info.num_cores, info.num_subcores
    nw = nc * ns
    ew = N_EDGES // nw
    n_chunks = ew // C

    @functools.partial(
        pl.kernel,
        out_type=jax.ShapeDtypeStruct((N_EDGES,), jnp.float32),
        mesh=plsc.VectorSubcoreMesh(core_axis_name="c", subcore_axis_name="s"),
        scratch_types=[
            pltpu.VMEM((ew,), jnp.int32),
            pltpu.VMEM((ew,), jnp.int32),
            pltpu.VMEM((C, D), jnp.float32),
            pltpu.VMEM((C, D), jnp.float32),
            pltpu.VMEM((C, D), jnp.float32),
            pltpu.VMEM((C, D), jnp.float32),
            pltpu.VMEM((ew,), jnp.float32),
            pltpu.SemaphoreType.DMA,
            pltpu.SemaphoreType.DMA,
        ],
    )
    def k(h_ref, src_ref, dst_ref, out_ref,
          idx_s, idx_d, rs0, rs1, rd0, rd1, scores, sem0, sem1):
        wid = lax.axis_index("s") * nc + lax.axis_index("c")
        ebase = pl.multiple_of(wid * ew, 8)
        pltpu.sync_copy(src_ref.at[pl.ds(ebase, ew)], idx_s)
        pltpu.sync_copy(dst_ref.at[pl.ds(ebase, ew)], idx_d)

        rows_s, rows_d, sems = [rs0, rs1], [rd0, rd1], [sem0, sem1]
        lane = lax.iota(jnp.int32, L)
        rot = [(lane + k) & (L - 1) for k in (8, 4, 2, 1)]

        def fire(ch, b):
            off = pl.multiple_of(ch * C, 8)
            pltpu.async_copy(h_ref.at[idx_s.at[pl.ds(off, C)]],
                             rows_s[b], sems[b])
            pltpu.async_copy(h_ref.at[idx_d.at[pl.ds(off, C)]],
                             rows_d[b], sems[b])

        def drain(b):
            pltpu.make_async_copy(h_ref.at[idx_s.at[pl.ds(0, C)]],
                                  rows_s[b], sems[b]).wait()
            pltpu.make_async_copy(h_ref.at[idx_d.at[pl.ds(0, C)]],
                                  rows_d[b], sems[b]).wait()

        def compute(ch, b):
            rs, rd = rows_s[b], rows_d[b]
            for eg in range(C // L):
                acc = jnp.zeros((L,), jnp.float32)
                for e in range(L):
                    ei = eg * L + e
                    p = rs[ei, pl.ds(0, L)] * rd[ei, pl.ds(0, L)]
                    for j in range(1, D // L):
                        p = p + (rs[ei, pl.ds(j * L, L)]
                                 * rd[ei, pl.ds(j * L, L)])
                    for r in rot:
                        p = p + _lane_shuffle(p, r)
                    acc = jnp.where(lane == e, p, acc)
                scores[pl.ds(ch * C + eg * L, L)] = acc

        fire(0, 0)

        def body(gg, carry):
            for b in range(2):
                ch = 2 * gg + b
                fire(ch + 1, 1 - b)
                drain(b)
                compute(ch, b)
            return carry

        lax.fori_loop(0, n_chunks // 2, body, 0)
        drain(0)
        compute(n_chunks - 1, 0)
        pltpu.sync_copy(scores, out_ref.at[pl.ds(ebase, ew)])

    return k(h, src, dst)


def kernel(h, edge_index):
    ei = edge_index.astype(jnp.int32)
    scores = _edge_dot(h, ei[0], ei[1])
    return scores.reshape(N_EDGES, 1)
